# initial kernel scaffold (unmeasured)
import jax
import jax.numpy as jnp
from jax import lax
from jax.experimental import pallas as pl
from jax.experimental.pallas import tpu as pltpu

N_Z = 4
N_HOPS = N_Z - 1
E_LOCAL = 2


def kernel(x, assign, W1, W2):
    t, d = x.shape
    _, _, f = W1.shape

    xbf = x.astype(jnp.bfloat16)
    oh = (assign[:, None] == jnp.arange(128, dtype=assign.dtype)[None, :])
    oh = oh.astype(jnp.bfloat16)
    w1 = W1.astype(jnp.bfloat16)
    w2 = W2.astype(jnp.bfloat16)

    def body(x_ref, oh_ref, w1_ref, w2_ref, out_ref,
             xg, ohg, rs_send, rs_recv,
             agx_s, agx_r, ago_s, ago_r, rs_s, rs_r):
        my_x = lax.axis_index("x")
        my_y = lax.axis_index("y")
        my_z = lax.axis_index("z")
        left = lax.rem(my_z + N_Z - 1, N_Z)
        right = lax.rem(my_z + 1, N_Z)

        barrier = pltpu.get_barrier_semaphore()
        for nbr in (left, right):
            pl.semaphore_signal(
                barrier, inc=1,
                device_id=(my_x, my_y, nbr),
                device_id_type=pl.DeviceIdType.MESH,
            )
        pl.semaphore_wait(barrier, 2)

        xg[0, :, :] = x_ref[:, :]
        ohg[0, :, :] = oh_ref[:, :]

        def start_hop(s):
            rx = pltpu.make_async_remote_copy(
                src_ref=xg.at[s], dst_ref=xg.at[s + 1],
                send_sem=agx_s.at[s], recv_sem=agx_r.at[s],
                device_id=(my_x, my_y, right),
                device_id_type=pl.DeviceIdType.MESH,
            )
            ro = pltpu.make_async_remote_copy(
                src_ref=ohg.at[s], dst_ref=ohg.at[s + 1],
                send_sem=ago_s.at[s], recv_sem=ago_r.at[s],
                device_id=(my_x, my_y, right),
                device_id_type=pl.DeviceIdType.MESH,
            )
            rx.start()
            ro.start()
            return rx, ro

        lane = lax.broadcasted_iota(jnp.int32, (1, 128), 1)

        def partial(slot):
            xc = xg[slot]
            ohc = ohg[slot]
            acc = None
            for j in range(E_LOCAL):
                e = E_LOCAL * my_z + j
                sel = (lane == e).astype(jnp.bfloat16)
                m = jnp.sum(ohc * sel, axis=1, keepdims=True)
                xm = xc * m
                h = jnp.dot(xm, w1_ref[j], preferred_element_type=jnp.float32)
                hb = jnp.maximum(h, 0.0).astype(jnp.bfloat16)
                o = jnp.dot(hb, w2_ref[j], preferred_element_type=jnp.float32)
                acc = o if acc is None else acc + o
            return acc

        hop = start_hop(0)
        out_ref[:, :] = partial(0)

        for s in range(N_HOPS):
            hop[0].wait()
            hop[1].wait()
            if s + 1 < N_HOPS:
                hop = start_hop(s + 1)
            p = partial(s + 1)
            if s > 0:
                p = p + rs_recv[s - 1].astype(jnp.float32)
            rs_send[:, :] = p.astype(jnp.bfloat16)
            rs_rdma = pltpu.make_async_remote_copy(
                src_ref=rs_send, dst_ref=rs_recv.at[s],
                send_sem=rs_s.at[s], recv_sem=rs_r.at[s],
                device_id=(my_x, my_y, right),
                device_id_type=pl.DeviceIdType.MESH,
            )
            rs_rdma.start()
            rs_rdma.wait()

        out_ref[:, :] = out_ref[:, :] + rs_recv[N_HOPS - 1].astype(jnp.float32)

    out_shape = jax.ShapeDtypeStruct((t, d), jnp.float32)
    return pl.pallas_call(
        body,
        out_shape=out_shape,
        in_specs=[pl.BlockSpec(memory_space=pltpu.VMEM)] * 4,
        out_specs=pl.BlockSpec(memory_space=pltpu.VMEM),
        scratch_shapes=[
            pltpu.VMEM((N_Z, t, d), jnp.bfloat16),
            pltpu.VMEM((N_Z, t, 128), jnp.bfloat16),
            pltpu.VMEM((t, d), jnp.bfloat16),
            pltpu.VMEM((N_HOPS, t, d), jnp.bfloat16),
            pltpu.SemaphoreType.DMA((N_HOPS,)),
            pltpu.SemaphoreType.DMA((N_HOPS,)),
            pltpu.SemaphoreType.DMA((N_HOPS,)),
            pltpu.SemaphoreType.DMA((N_HOPS,)),
            pltpu.SemaphoreType.DMA((N_HOPS,)),
            pltpu.SemaphoreType.DMA((N_HOPS,)),
        ],
        compiler_params=pltpu.CompilerParams(collective_id=0),
    )(xbf, oh, w1, w2)


# baseline (device time: 197870 ns/iter reference)
import jax
import jax.numpy as jnp
from jax import lax
from jax.experimental import pallas as pl
from jax.experimental.pallas import tpu as pltpu

N_Z = 4
N_HOPS = N_Z - 1
E_LOCAL = 2


def kernel(x, assign, W1, W2):
    t, d = x.shape
    _, _, f = W1.shape

    xbf = x.astype(jnp.bfloat16)
    oh = (assign[:, None] == jnp.arange(128, dtype=assign.dtype)[None, :])
    oh = oh.astype(jnp.bfloat16)
    w1 = W1.astype(jnp.bfloat16)
    w2 = W2.astype(jnp.bfloat16)

    def body(x_ref, oh_ref, w1_ref, w2_ref, out_ref,
             xg, ohg, rs_send, rs_recv,
             agx_s, agx_r, ago_s, ago_r, rs_s, rs_r):
        my_x = lax.axis_index("x")
        my_y = lax.axis_index("y")
        my_z = lax.axis_index("z")
        left = lax.rem(my_z + N_Z - 1, N_Z)
        right = lax.rem(my_z + 1, N_Z)

        barrier = pltpu.get_barrier_semaphore()
        for nbr in (left, right):
            pl.semaphore_signal(
                barrier, inc=1,
                device_id=(my_x, my_y, nbr),
                device_id_type=pl.DeviceIdType.MESH,
            )
        pl.semaphore_wait(barrier, 2)

        xg[0, :, :] = x_ref[:, :]
        ohg[0, :, :] = oh_ref[:, :]

        def start_hop(s):
            rx = pltpu.make_async_remote_copy(
                src_ref=xg.at[s], dst_ref=xg.at[s + 1],
                send_sem=agx_s.at[s], recv_sem=agx_r.at[s],
                device_id=(my_x, my_y, right),
                device_id_type=pl.DeviceIdType.MESH,
            )
            ro = pltpu.make_async_remote_copy(
                src_ref=ohg.at[s], dst_ref=ohg.at[s + 1],
                send_sem=ago_s.at[s], recv_sem=ago_r.at[s],
                device_id=(my_x, my_y, right),
                device_id_type=pl.DeviceIdType.MESH,
            )
            rx.start()
            ro.start()
            return rx, ro

        lane = lax.broadcasted_iota(jnp.int32, (1, 128), 1)

        FT = 512

        def partial(slot):
            xc = xg[slot]
            ohc = ohg[slot]
            acc = jnp.zeros((t, d), jnp.float32)
            for j in range(E_LOCAL):
                e = E_LOCAL * my_z + j
                sel = (lane == e).astype(jnp.bfloat16)
                m = jnp.sum(ohc * sel, axis=1, keepdims=True)
                xm = xc * m
                for ft in range(0, f, FT):
                    h = jnp.dot(xm, w1_ref[j, :, ft:ft + FT],
                                preferred_element_type=jnp.float32)
                    hb = jnp.maximum(h, 0.0).astype(jnp.bfloat16)
                    acc = acc + jnp.dot(hb, w2_ref[j, ft:ft + FT, :],
                                        preferred_element_type=jnp.float32)
            return acc

        hop = start_hop(0)
        out_ref[:, :] = partial(0)

        for s in range(N_HOPS):
            hop[0].wait()
            hop[1].wait()
            if s + 1 < N_HOPS:
                hop = start_hop(s + 1)
            p = partial(s + 1)
            if s > 0:
                p = p + rs_recv[s - 1].astype(jnp.float32)
            rs_send[:, :] = p.astype(jnp.bfloat16)
            rs_rdma = pltpu.make_async_remote_copy(
                src_ref=rs_send, dst_ref=rs_recv.at[s],
                send_sem=rs_s.at[s], recv_sem=rs_r.at[s],
                device_id=(my_x, my_y, right),
                device_id_type=pl.DeviceIdType.MESH,
            )
            rs_rdma.start()
            rs_rdma.wait()

        out_ref[:, :] = out_ref[:, :] + rs_recv[N_HOPS - 1].astype(jnp.float32)

    out_shape = jax.ShapeDtypeStruct((t, d), jnp.float32)
    return pl.pallas_call(
        body,
        out_shape=out_shape,
        in_specs=[pl.BlockSpec(memory_space=pltpu.VMEM)] * 4,
        out_specs=pl.BlockSpec(memory_space=pltpu.VMEM),
        scratch_shapes=[
            pltpu.VMEM((N_Z, t, d), jnp.bfloat16),
            pltpu.VMEM((N_Z, t, 128), jnp.bfloat16),
            pltpu.VMEM((t, d), jnp.bfloat16),
            pltpu.VMEM((N_HOPS, t, d), jnp.bfloat16),
            pltpu.SemaphoreType.DMA((N_HOPS,)),
            pltpu.SemaphoreType.DMA((N_HOPS,)),
            pltpu.SemaphoreType.DMA((N_HOPS,)),
            pltpu.SemaphoreType.DMA((N_HOPS,)),
            pltpu.SemaphoreType.DMA((N_HOPS,)),
            pltpu.SemaphoreType.DMA((N_HOPS,)),
        ],
        compiler_params=pltpu.CompilerParams(collective_id=0),
    )(xbf, oh, w1, w2)


# device time: 187544 ns/iter; 1.0551x vs baseline; 1.0551x over previous
import jax
import jax.numpy as jnp
from jax import lax
from jax.experimental import pallas as pl
from jax.experimental.pallas import tpu as pltpu

N_Z = 4
N_HOPS = N_Z - 1
E_LOCAL = 2
OH = 128


def kernel(x, assign, W1, W2):
    t, d = x.shape
    _, _, f = W1.shape

    xbf = x.astype(jnp.bfloat16)
    oh = (assign[:, None] == jnp.arange(OH, dtype=assign.dtype)[None, :])
    slab = jnp.concatenate([xbf, oh.astype(jnp.bfloat16)], axis=1)
    w1 = W1.astype(jnp.bfloat16)
    w2 = W2.astype(jnp.bfloat16)

    def body(slab_ref, w1_ref, w2_ref, out_ref,
             xg, rs_send, rs_recv, pacc,
             ag_s, ag_r, rs_s, rs_r):
        my_x = lax.axis_index("x")
        my_y = lax.axis_index("y")
        my_z = lax.axis_index("z")
        left = lax.rem(my_z + N_Z - 1, N_Z)
        right = lax.rem(my_z + 1, N_Z)

        barrier = pltpu.get_barrier_semaphore()
        for nbr in (left, right):
            pl.semaphore_signal(
                barrier, inc=1,
                device_id=(my_x, my_y, nbr),
                device_id_type=pl.DeviceIdType.MESH,
            )
        pl.semaphore_wait(barrier, 2)

        xg[0, :, :] = slab_ref[:, :]

        def start_hop(s):
            r = pltpu.make_async_remote_copy(
                src_ref=xg.at[s], dst_ref=xg.at[s + 1],
                send_sem=ag_s.at[s], recv_sem=ag_r.at[s],
                device_id=(my_x, my_y, right),
                device_id_type=pl.DeviceIdType.MESH,
            )
            r.start()
            return r

        lane = lax.broadcasted_iota(jnp.int32, (1, OH), 1)

        FT = 512

        def expert_contrib(slot, j, acc, first):
            chunk = xg[slot]
            xc = chunk[:, :d]
            ohc = chunk[:, d:]
            e = E_LOCAL * my_z + j
            sel = (lane == e).astype(jnp.bfloat16)
            m = jnp.sum(ohc * sel, axis=1, keepdims=True)
            xm = xc * m
            for k, ft in enumerate(range(0, f, FT)):
                h = jnp.dot(xm, w1_ref[j, :, ft:ft + FT],
                            preferred_element_type=jnp.float32)
                hb = jnp.maximum(h, 0.0).astype(jnp.bfloat16)
                c = jnp.dot(hb, w2_ref[j, ft:ft + FT, :],
                            preferred_element_type=jnp.float32)
                if first and k == 0:
                    acc[:, :] = c
                else:
                    acc[:, :] = acc[:, :] + c

        ag = [start_hop(0)]
        expert_contrib(0, 0, out_ref, first=True)

        rs = []
        for s in range(N_HOPS):
            ag[s].wait_recv()
            if s + 1 < N_HOPS:
                ag.append(start_hop(s + 1))
            expert_contrib(s + 1, 0, pacc, first=True)
            expert_contrib(s + 1, 1, pacc, first=False)
            if s > 0:
                rs[s - 1].wait_recv()
                p = pacc[:, :] + rs_recv[s - 1].astype(jnp.float32)
            else:
                p = pacc[:, :]
            slot = s % 2
            if s >= 2:
                rs[s - 2].wait_send()
            rs_send[slot, :, :] = p.astype(jnp.bfloat16)
            r = pltpu.make_async_remote_copy(
                src_ref=rs_send.at[slot], dst_ref=rs_recv.at[s],
                send_sem=rs_s.at[s], recv_sem=rs_r.at[s],
                device_id=(my_x, my_y, right),
                device_id_type=pl.DeviceIdType.MESH,
            )
            r.start()
            rs.append(r)

        expert_contrib(0, 1, out_ref, first=False)

        rs[N_HOPS - 1].wait_recv()
        out_ref[:, :] = (out_ref[:, :]
                         + rs_recv[N_HOPS - 1].astype(jnp.float32))

        for r in ag:
            r.wait_send()
        for r in rs[max(0, N_HOPS - 2):]:
            r.wait_send()

    out_shape = jax.ShapeDtypeStruct((t, d), jnp.float32)
    return pl.pallas_call(
        body,
        out_shape=out_shape,
        in_specs=[pl.BlockSpec(memory_space=pltpu.VMEM)] * 3,
        out_specs=pl.BlockSpec(memory_space=pltpu.VMEM),
        scratch_shapes=[
            pltpu.VMEM((N_Z, t, d + OH), jnp.bfloat16),
            pltpu.VMEM((2, t, d), jnp.bfloat16),
            pltpu.VMEM((N_HOPS, t, d), jnp.bfloat16),
            pltpu.VMEM((t, d), jnp.float32),
            pltpu.SemaphoreType.DMA((N_HOPS,)),
            pltpu.SemaphoreType.DMA((N_HOPS,)),
            pltpu.SemaphoreType.DMA((N_HOPS,)),
            pltpu.SemaphoreType.DMA((N_HOPS,)),
        ],
        compiler_params=pltpu.CompilerParams(
            collective_id=0,
            vmem_limit_bytes=100 * 1024 * 1024,
        ),
    )(slab, w1, w2)
